# Initial kernel scaffold; baseline (speedup 1.0000x reference)
import jax
import jax.numpy as jnp
from jax.experimental import pallas as pl


def kernel(y_hat, y):
    raise NotImplementedError("write your pallas kernel here")


# speedup vs baseline: 2.8591x; 2.8591x over previous
import jax
import jax.numpy as jnp
from jax.experimental import pallas as pl
from jax.experimental.pallas import tpu as pltpu

_SCALE = 30.0
_R2 = 0.7071067811865476


def _block_body(yh_ref, y_ref, out_ref):
    i = pl.program_id(0)
    nsteps = pl.num_programs(0)

    yh = yh_ref[...]
    yv = y_ref[...]

    cols = jax.lax.broadcasted_iota(jnp.int32, yh.shape, 1)
    mask = cols == yv

    c = jnp.sum(jnp.where(mask, yh, 0.0), axis=1, keepdims=True)
    c = jnp.clip(c, -1.0, 1.0)

    c2 = c * c
    cos4 = 8.0 * c2 * c2 - 8.0 * c2 + 1.0
    k = (
        (c <= _R2).astype(jnp.int32)
        + (c <= 0.0).astype(jnp.int32)
        + (c <= -_R2).astype(jnp.int32)
    )
    co = jnp.where((k & 1) == 1, -1.0, 1.0)
    psi = co * cos4 - 2.0 * k.astype(jnp.float32)

    fc = jnp.where(mask, psi, yh) * _SCALE
    m = jnp.max(fc, axis=1, keepdims=True)
    s = jnp.sum(jnp.exp(fc - m), axis=1, keepdims=True)
    lse = m + jnp.log(s)

    part = jnp.sum(lse - _SCALE * psi)

    @pl.when(i == 0)
    def _init():
        out_ref[0, 0] = 0.0

    out_ref[0, 0] += part

    @pl.when(i == nsteps - 1)
    def _final():
        out_ref[0, 0] = out_ref[0, 0] * (1.0 / (nsteps * yh.shape[0]))


def kernel(y_hat, y):
    n, num_class = y_hat.shape
    blk = 256
    grid = n // blk
    y2 = y.reshape(n, 1)

    out = pl.pallas_call(
        _block_body,
        grid=(grid,),
        in_specs=[
            pl.BlockSpec((blk, num_class), lambda i: (i, 0)),
            pl.BlockSpec((blk, 1), lambda i: (i, 0)),
        ],
        out_specs=pl.BlockSpec(
            (1, 1), lambda i: (0, 0), memory_space=pltpu.SMEM
        ),
        out_shape=jax.ShapeDtypeStruct((1, 1), jnp.float32),
    )(y_hat, y2)
    return out[0, 0]


# speedup vs baseline: 2.9207x; 1.0215x over previous
import jax
import jax.numpy as jnp
from jax.experimental import pallas as pl
from jax.experimental.pallas import tpu as pltpu

_SCALE = 30.0
_R2 = 0.7071067811865476


_LOG2E = 1.4426950408889634


def _block_body(yh_ref, y_ref, out_ref):
    i = pl.program_id(0)
    nsteps = pl.num_programs(0)

    yh = yh_ref[...]
    yv = y_ref[...]

    cols = jax.lax.broadcasted_iota(jnp.int32, yh.shape, 1)
    mask = cols == yv

    c = jnp.sum(jnp.where(mask, yh, 0.0), axis=1, keepdims=True)
    c = jnp.clip(c, -1.0, 1.0)

    c2 = c * c
    cos4 = 8.0 * c2 * c2 - 8.0 * c2 + 1.0
    k = (
        (c <= _R2).astype(jnp.int32)
        + (c <= 0.0).astype(jnp.int32)
        + (c <= -_R2).astype(jnp.int32)
    )
    co = jnp.where((k & 1) == 1, -1.0, 1.0)
    psi = co * cos4 - 2.0 * k.astype(jnp.float32)

    m0 = jnp.max(yh, axis=1, keepdims=True)
    M = jnp.maximum(_SCALE * m0, _SCALE * psi)
    a = _SCALE * _LOG2E
    b = M * _LOG2E
    s0 = jnp.sum(jnp.exp2(yh * a - b), axis=1, keepdims=True)
    s = s0 - jnp.exp2(c * a - b) + jnp.exp2(psi * a - b)
    lse = M + jnp.log(s)

    part = jnp.sum(lse - _SCALE * psi)

    @pl.when(i == 0)
    def _init():
        out_ref[0, 0] = 0.0

    out_ref[0, 0] += part

    @pl.when(i == nsteps - 1)
    def _final():
        out_ref[0, 0] = out_ref[0, 0] * (1.0 / (nsteps * yh.shape[0]))


def kernel(y_hat, y):
    n, num_class = y_hat.shape
    blk = 256
    grid = n // blk
    y2 = y.reshape(n, 1)

    out = pl.pallas_call(
        _block_body,
        grid=(grid,),
        in_specs=[
            pl.BlockSpec((blk, num_class), lambda i: (i, 0)),
            pl.BlockSpec((blk, 1), lambda i: (i, 0)),
        ],
        out_specs=pl.BlockSpec(
            (1, 1), lambda i: (0, 0), memory_space=pltpu.SMEM
        ),
        out_shape=jax.ShapeDtypeStruct((1, 1), jnp.float32),
    )(y_hat, y2)
    return out[0, 0]


# speedup vs baseline: 3.6762x; 1.2587x over previous
import jax
import jax.numpy as jnp
from jax.experimental import pallas as pl
from jax.experimental.pallas import tpu as pltpu

_SCALE = 30.0
_R2 = 0.7071067811865476


_LOG2E = 1.4426950408889634


def _block_body(yh_ref, y_ref, out_ref):
    i = pl.program_id(0)
    nsteps = pl.num_programs(0)

    yh = yh_ref[...]
    yv = y_ref[...]

    cols = jax.lax.broadcasted_iota(jnp.int32, yh.shape, 1)
    mask = cols == yv

    c = jnp.sum(jnp.where(mask, yh, 0.0), axis=1, keepdims=True)
    c = jnp.clip(c, -1.0, 1.0)

    c2 = c * c
    cos4 = 8.0 * c2 * c2 - 8.0 * c2 + 1.0
    k = (
        (c <= _R2).astype(jnp.int32)
        + (c <= 0.0).astype(jnp.int32)
        + (c <= -_R2).astype(jnp.int32)
    )
    co = jnp.where((k & 1) == 1, -1.0, 1.0)
    psi = co * cos4 - 2.0 * k.astype(jnp.float32)

    m0 = jnp.max(yh, axis=1, keepdims=True)
    M = jnp.maximum(_SCALE * m0, _SCALE * psi)
    a = _SCALE * _LOG2E
    b = M * _LOG2E
    s0 = jnp.sum(jnp.exp2(yh * a - b), axis=1, keepdims=True)
    s = s0 - jnp.exp2(c * a - b) + jnp.exp2(psi * a - b)
    lse = M + jnp.log(s)

    part = jnp.sum(lse - _SCALE * psi)

    @pl.when(i == 0)
    def _init():
        out_ref[0, 0] = 0.0

    out_ref[0, 0] += part

    @pl.when(i == nsteps - 1)
    def _final():
        out_ref[0, 0] = out_ref[0, 0] * (1.0 / (nsteps * yh.shape[0]))


def kernel(y_hat, y):
    n, num_class = y_hat.shape
    blk = 1024
    grid = n // blk
    y2 = y.reshape(n, 1)

    out = pl.pallas_call(
        _block_body,
        grid=(grid,),
        in_specs=[
            pl.BlockSpec((blk, num_class), lambda i: (i, 0)),
            pl.BlockSpec((blk, 1), lambda i: (i, 0)),
        ],
        out_specs=pl.BlockSpec(
            (1, 1), lambda i: (0, 0), memory_space=pltpu.SMEM
        ),
        out_shape=jax.ShapeDtypeStruct((1, 1), jnp.float32),
    )(y_hat, y2)
    return out[0, 0]
